# baseline (speedup 1.0000x reference)
import functools

import jax
import jax.numpy as jnp
from jax import lax
from jax.experimental import pallas as pl
from jax.experimental.pallas import tpu as pltpu
from jax.experimental.pallas import tpu_sc as plsc

DIM = 64
RANK = 8
IDX_CHUNK = 128


def _sc_gather(pid, base, lora_A, n_workers, num_cores):
    batch = pid.shape[0]
    b_per_w = batch // n_workers
    n_chunks = b_per_w // IDX_CHUNK
    mesh = plsc.VectorSubcoreMesh(core_axis_name="c", subcore_axis_name="s")

    @functools.partial(
        pl.kernel,
        mesh=mesh,
        out_type=[
            jax.ShapeDtypeStruct((batch, DIM), jnp.float32),
            jax.ShapeDtypeStruct((batch, RANK), jnp.float32),
        ],
        scratch_types=[
            pltpu.VMEM((n_chunks, IDX_CHUNK), jnp.int32),
            pltpu.VMEM((b_per_w, DIM), jnp.float32),
            pltpu.VMEM((b_per_w, RANK), jnp.float32),
            pltpu.SemaphoreType.DMA,
        ],
        compiler_params=pltpu.CompilerParams(use_tc_tiling_on_sc=False),
    )
    def gather_kernel(pid_hbm, base_hbm, la_hbm, out_rows, out_a,
                      idx_v, rows_v, a_v, sem):
        wid = lax.axis_index("s") * num_cores + lax.axis_index("c")
        base_off = wid * b_per_w
        for j in range(n_chunks):
            pltpu.sync_copy(
                pid_hbm.at[pl.ds(base_off + j * IDX_CHUNK, IDX_CHUNK)],
                idx_v.at[j])
        copies = []
        for j in range(n_chunks):
            copies.append(pltpu.async_copy(
                base_hbm.at[idx_v.at[j]],
                rows_v.at[pl.ds(j * IDX_CHUNK, IDX_CHUNK)], sem))
            copies.append(pltpu.async_copy(
                la_hbm.at[idx_v.at[j]],
                a_v.at[pl.ds(j * IDX_CHUNK, IDX_CHUNK)], sem))
        for cp in copies:
            cp.wait()
        pltpu.sync_copy(rows_v, out_rows.at[pl.ds(base_off, b_per_w)])
        pltpu.sync_copy(a_v, out_a.at[pl.ds(base_off, b_per_w)])

    return gather_kernel(pid, base, lora_A)


def _tc_fuse(rows, a, lora_B_w):
    batch = rows.shape[0]
    block = 2048
    grid = batch // block

    def body(rows_ref, a_ref, bw_ref, o_ref):
        delta = lax.dot_general(
            a_ref[...], bw_ref[...],
            dimension_numbers=(((1,), (1,)), ((), ())),
            preferred_element_type=jnp.float32)
        o_ref[...] = rows_ref[...] + delta

    return pl.pallas_call(
        body,
        grid=(grid,),
        in_specs=[
            pl.BlockSpec((block, DIM), lambda i: (i, 0)),
            pl.BlockSpec((block, RANK), lambda i: (i, 0)),
            pl.BlockSpec((DIM, RANK), lambda i: (0, 0)),
        ],
        out_specs=pl.BlockSpec((block, DIM), lambda i: (i, 0)),
        out_shape=jax.ShapeDtypeStruct((batch, DIM), jnp.float32),
    )(rows, a, lora_B_w)


def kernel(pid, base, lora_A, lora_B_w):
    info = plsc.get_sparse_core_info()
    n_workers = info.num_cores * info.num_subcores
    pid = pid.astype(jnp.int32)
    rows, a = _sc_gather(pid, base, lora_A, n_workers, info.num_cores)
    return _tc_fuse(rows, a, lora_B_w)


# speedup vs baseline: 1.5171x; 1.5171x over previous
import functools

import jax
import jax.numpy as jnp
from jax import lax
from jax.experimental import pallas as pl
from jax.experimental.pallas import tpu as pltpu
from jax.experimental.pallas import tpu_sc as plsc

DIM = 64
RANK = 8
FIRE = 16


def _sc_gather(pid, base, lora_A, n_workers, num_cores):
    batch = pid.shape[0]
    b_per_w = batch // n_workers
    n_pass = 2
    b_chunk = b_per_w // n_pass
    mesh = plsc.VectorSubcoreMesh(core_axis_name="c", subcore_axis_name="s")

    @functools.partial(
        pl.kernel,
        mesh=mesh,
        out_type=[
            jax.ShapeDtypeStruct((batch, DIM), jnp.float32),
            jax.ShapeDtypeStruct((batch, RANK), jnp.float32),
        ],
        scratch_types=[
            pltpu.VMEM((b_per_w,), jnp.int32),
            pltpu.VMEM((b_chunk, DIM), jnp.float32),
            pltpu.VMEM((b_chunk, RANK), jnp.float32),
            pltpu.SemaphoreType.DMA,
            pltpu.SemaphoreType.DMA,
        ],
    )
    def gather_kernel(pid_hbm, base_hbm, la_hbm, out_rows, out_a,
                      idx_flat, rows_v, a_v, sem, sem2):
        wid = lax.axis_index("s") * num_cores + lax.axis_index("c")
        base_off = wid * b_per_w
        pltpu.sync_copy(pid_hbm.at[pl.ds(base_off, b_per_w)], idx_flat)
        for p in range(n_pass):
            def row_chunk(c, _):
                vec = idx_flat[pl.ds(p * b_chunk + c * FIRE, FIRE)]
                copies = []
                for t in range(FIRE):
                    i = c * FIRE + t
                    copies.append(pltpu.async_copy(
                        base_hbm.at[vec[t]], rows_v.at[i], sem))
                    copies.append(pltpu.async_copy(
                        la_hbm.at[vec[t]], a_v.at[i], sem2))
                for cp in copies:
                    cp.wait()
                return 0
            lax.fori_loop(0, b_chunk // FIRE, row_chunk, 0)
            pltpu.sync_copy(
                rows_v,
                out_rows.at[pl.ds(base_off + p * b_chunk, b_chunk)])
            pltpu.sync_copy(
                a_v, out_a.at[pl.ds(base_off + p * b_chunk, b_chunk)])

    return gather_kernel(pid, base, lora_A)


def _tc_fuse(rows, a, lora_B_w):
    batch = rows.shape[0]
    block = 2048
    grid = batch // block

    def body(rows_ref, a_ref, bw_ref, o_ref):
        delta = lax.dot_general(
            a_ref[...], bw_ref[...],
            dimension_numbers=(((1,), (1,)), ((), ())),
            preferred_element_type=jnp.float32)
        o_ref[...] = rows_ref[...] + delta

    return pl.pallas_call(
        body,
        grid=(grid,),
        in_specs=[
            pl.BlockSpec((block, DIM), lambda i: (i, 0)),
            pl.BlockSpec((block, RANK), lambda i: (i, 0)),
            pl.BlockSpec((DIM, RANK), lambda i: (0, 0)),
        ],
        out_specs=pl.BlockSpec((block, DIM), lambda i: (i, 0)),
        out_shape=jax.ShapeDtypeStruct((batch, DIM), jnp.float32),
    )(rows, a, lora_B_w)


def kernel(pid, base, lora_A, lora_B_w):
    info = plsc.get_sparse_core_info()
    n_workers = info.num_cores * info.num_subcores
    pid = pid.astype(jnp.int32)
    rows, a = _sc_gather(pid, base, lora_A, n_workers, info.num_cores)
    return _tc_fuse(rows, a, lora_B_w)


# speedup vs baseline: 2.3761x; 1.5663x over previous
import functools

import jax
import jax.numpy as jnp
from jax import lax
from jax.experimental import pallas as pl
from jax.experimental.pallas import tpu as pltpu
from jax.experimental.pallas import tpu_sc as plsc

DIM = 64
RANK = 8
LANES = 16
FIRE = 16


def _sc_gather_lora(pid, lat, n_workers, num_cores):
    batch = pid.shape[0]
    b_per_w = batch // n_workers
    n_grp = b_per_w // LANES
    mesh = plsc.VectorSubcoreMesh(core_axis_name="c", subcore_axis_name="s")

    @functools.partial(
        pl.kernel,
        mesh=mesh,
        out_type=jax.ShapeDtypeStruct((batch * RANK,), jnp.float32),
        scratch_types=[
            pltpu.VMEM((b_per_w,), jnp.int32),
            pltpu.VMEM((LANES * RANK, 128), jnp.float32),
            pltpu.VMEM((b_per_w * RANK,), jnp.float32),
            pltpu.SemaphoreType.DMA,
        ],
        compiler_params=pltpu.CompilerParams(needs_layout_passes=False),
    )
    def lora_kernel(pid_hbm, lat_hbm, out_a, idx_flat, block_v, a_flat, sem):
        wid = lax.axis_index("s") * num_cores + lax.axis_index("c")
        base_off = wid * b_per_w
        pltpu.sync_copy(pid_hbm.at[pl.ds(base_off, b_per_w)], idx_flat)
        iota = lax.iota(jnp.int32, LANES)
        lane8 = iota < 8

        def group(g, _):
            vec = idx_flat[pl.ds(g * LANES, LANES)]
            copies = []
            for t in range(FIRE):
                off = pl.multiple_of((vec[t] >> 7) * 128, 128)
                copies.append(pltpu.async_copy(
                    lat_hbm.at[:, pl.ds(off, 128)],
                    block_v.at[pl.ds(RANK * t, RANK), :], sem))
            for cp in copies:
                cp.wait()
            lvec = vec & 127
            for t in range(FIRE):
                col = jnp.zeros((LANES,), jnp.int32) + lvec[t]
                row = RANK * t + iota
                x = plsc.load_gather(block_v, [row, col], mask=lane8)
                dst = (g * LANES + t) * RANK + iota
                plsc.store_scatter(a_flat, [dst], x, mask=lane8)
            return 0
        lax.fori_loop(0, n_grp, group, 0)
        pltpu.sync_copy(a_flat, out_a.at[pl.ds(base_off * RANK,
                                               b_per_w * RANK)])

    return lora_kernel(pid, lat)


def _sc_gather_base(pid, base, n_workers, num_cores):
    batch = pid.shape[0]
    b_per_w = batch // n_workers
    n_pass = 2
    b_chunk = b_per_w // n_pass
    mesh = plsc.VectorSubcoreMesh(core_axis_name="c", subcore_axis_name="s")

    @functools.partial(
        pl.kernel,
        mesh=mesh,
        out_type=jax.ShapeDtypeStruct((batch, DIM), jnp.float32),
        scratch_types=[
            pltpu.VMEM((b_per_w,), jnp.int32),
            pltpu.VMEM((b_chunk, DIM), jnp.float32),
            pltpu.SemaphoreType.DMA,
        ],
    )
    def base_kernel(pid_hbm, base_hbm, out_rows, idx_flat, rows_v, sem):
        wid = lax.axis_index("s") * num_cores + lax.axis_index("c")
        base_off = wid * b_per_w
        pltpu.sync_copy(pid_hbm.at[pl.ds(base_off, b_per_w)], idx_flat)
        for p in range(n_pass):
            def row_chunk(c, _):
                vec = idx_flat[pl.ds(p * b_chunk + c * FIRE, FIRE)]
                copies = []
                for t in range(FIRE):
                    copies.append(pltpu.async_copy(
                        base_hbm.at[vec[t]], rows_v.at[c * FIRE + t], sem))
                for cp in copies:
                    cp.wait()
                return 0
            lax.fori_loop(0, b_chunk // FIRE, row_chunk, 0)
            pltpu.sync_copy(
                rows_v,
                out_rows.at[pl.ds(base_off + p * b_chunk, b_chunk)])

    return base_kernel(pid, base)


def _tc_fuse(rows, a, lora_B_w):
    batch = rows.shape[0]
    block = 2048
    grid = batch // block

    def body(rows_ref, a_ref, bw_ref, o_ref):
        delta = lax.dot_general(
            a_ref[...], bw_ref[...],
            dimension_numbers=(((1,), (1,)), ((), ())),
            preferred_element_type=jnp.float32)
        o_ref[...] = rows_ref[...] + delta

    return pl.pallas_call(
        body,
        grid=(grid,),
        in_specs=[
            pl.BlockSpec((block, DIM), lambda i: (i, 0)),
            pl.BlockSpec((block, RANK), lambda i: (i, 0)),
            pl.BlockSpec((DIM, RANK), lambda i: (0, 0)),
        ],
        out_specs=pl.BlockSpec((block, DIM), lambda i: (i, 0)),
        out_shape=jax.ShapeDtypeStruct((batch, DIM), jnp.float32),
    )(rows, a, lora_B_w)


def kernel(pid, base, lora_A, lora_B_w):
    info = plsc.get_sparse_core_info()
    n_workers = info.num_cores * info.num_subcores
    pid = pid.astype(jnp.int32)
    batch = pid.shape[0]
    a_flat = _sc_gather_lora(pid, lora_A.T, n_workers, info.num_cores)
    rows = _sc_gather_base(pid, base, n_workers, info.num_cores)
    a = a_flat.reshape(batch, RANK)
    return _tc_fuse(rows, a, lora_B_w)


# speedup vs baseline: 2.5694x; 1.0814x over previous
import functools

import jax
import jax.numpy as jnp
from jax import lax
from jax.experimental import pallas as pl
from jax.experimental.pallas import tpu as pltpu
from jax.experimental.pallas import tpu_sc as plsc

DIM = 64
RANK = 8
LANES = 16
FIRE = 16


def _sc_gather_lora(pid, lat, n_workers, num_cores):
    batch = pid.shape[0]
    b_per_w = batch // n_workers
    n_grp = b_per_w // LANES
    mesh = plsc.VectorSubcoreMesh(core_axis_name="c", subcore_axis_name="s")

    @functools.partial(
        pl.kernel,
        mesh=mesh,
        out_type=jax.ShapeDtypeStruct((batch * RANK,), jnp.float32),
        scratch_types=[
            pltpu.VMEM((b_per_w,), jnp.int32),
            pltpu.VMEM((LANES * RANK, 128), jnp.float32),
            pltpu.VMEM((b_per_w * RANK,), jnp.float32),
            pltpu.SemaphoreType.DMA,
        ],
        compiler_params=pltpu.CompilerParams(needs_layout_passes=False),
    )
    def lora_kernel(pid_hbm, lat_hbm, out_a, idx_flat, block_v, a_flat, sem):
        wid = lax.axis_index("s") * num_cores + lax.axis_index("c")
        base_off = wid * b_per_w
        pltpu.sync_copy(pid_hbm.at[pl.ds(base_off, b_per_w)], idx_flat)
        iota = lax.iota(jnp.int32, LANES)
        lane8 = iota < 8

        def group(g, _):
            vec = idx_flat[pl.ds(g * LANES, LANES)]
            copies = []
            for t in range(FIRE):
                off = pl.multiple_of((vec[t] >> 7) * 128, 128)
                copies.append(pltpu.async_copy(
                    lat_hbm.at[:, pl.ds(off, 128)],
                    block_v.at[pl.ds(RANK * t, RANK), :], sem))
            for cp in copies:
                cp.wait()
            lvec = vec & 127
            for t in range(FIRE):
                col = jnp.zeros((LANES,), jnp.int32) + lvec[t]
                row = RANK * t + iota
                x = plsc.load_gather(block_v, [row, col], mask=lane8)
                dst = (g * LANES + t) * RANK + iota
                plsc.store_scatter(a_flat, [dst], x, mask=lane8)
            return 0
        lax.fori_loop(0, n_grp, group, 0)
        pltpu.sync_copy(a_flat, out_a.at[pl.ds(base_off * RANK,
                                               b_per_w * RANK)])

    return lora_kernel(pid, lat)


def _tc_transpose(base_t):
    n = base_t.shape[1]
    block = 2048
    grid = (n + 2 * block - 1) // (2 * block)
    last_full = n // block - 1

    def body(x0_ref, x1_ref, o_ref):
        o_ref[:, 0:DIM] = x0_ref[...].T
        o_ref[:, DIM:2 * DIM] = x1_ref[...].T

    return pl.pallas_call(
        body,
        grid=(grid,),
        in_specs=[
            pl.BlockSpec((DIM, block), lambda i: (0, 2 * i)),
            pl.BlockSpec(
                (DIM, block),
                lambda i: (0, jnp.minimum(2 * i + 1, last_full))),
        ],
        out_specs=pl.BlockSpec((block, 2 * DIM), lambda i: (i, 0)),
        out_shape=jax.ShapeDtypeStruct((grid * block, 2 * DIM), jnp.float32),
    )(base_t, base_t)


def _sc_gather_base(pid, base_pairs, n_workers, num_cores):
    batch = pid.shape[0]
    b_per_w = batch // n_workers
    n_pass = 2
    b_chunk = b_per_w // n_pass
    n_chunk = b_chunk // 128
    n_grp = b_chunk // LANES
    mesh = plsc.VectorSubcoreMesh(core_axis_name="c", subcore_axis_name="s")

    @functools.partial(
        pl.kernel,
        mesh=mesh,
        out_type=jax.ShapeDtypeStruct((batch, DIM), jnp.float32),
        scratch_types=[
            pltpu.VMEM((b_per_w,), jnp.int32),
            pltpu.VMEM((n_chunk, 128), jnp.int32),
            pltpu.VMEM((b_chunk, 2 * DIM), jnp.float32),
            pltpu.VMEM((b_chunk, DIM), jnp.float32),
            pltpu.SemaphoreType.DMA,
        ],
        compiler_params=pltpu.CompilerParams(needs_layout_passes=False),
    )
    def base_kernel(pid_hbm, base_hbm, out_rows, idx_v, pair_v, stage_v,
                    rows_v, sem):
        wid = lax.axis_index("s") * num_cores + lax.axis_index("c")
        base_off = wid * b_per_w
        pltpu.sync_copy(pid_hbm.at[pl.ds(base_off, b_per_w)], idx_v)
        iota = lax.iota(jnp.int32, LANES)

        for p in range(n_pass):
            for g in range(n_grp):
                vec = idx_v[pl.ds(p * b_chunk + g * LANES, LANES)]
                row, col = divmod(g * LANES, 128)
                pair_v[row, pl.ds(col, LANES)] = (
                    ((vec >> 12) << 11) + (vec & 2047))

            copies = []
            for j in range(n_chunk):
                copies.append(pltpu.async_copy(
                    base_hbm.at[pair_v.at[j]],
                    stage_v.at[pl.ds(j * 128, 128)], sem))
            for cp in copies:
                cp.wait()

            def extract(g, _):
                vec = idx_v[pl.ds(p * b_chunk + g * LANES, LANES)]
                for t in range(LANES):
                    j = g * LANES + t
                    off = ((vec[t] >> 11) & 1) * DIM
                    row = jnp.zeros((LANES,), jnp.int32) + j
                    for q in range(DIM // LANES):
                        col = off + q * LANES + iota
                        x = plsc.load_gather(stage_v, [row, col])
                        plsc.store_scatter(rows_v, [row, q * LANES + iota], x)
                return 0
            lax.fori_loop(0, n_grp, extract, 0)
            pltpu.sync_copy(
                rows_v,
                out_rows.at[pl.ds(base_off + p * b_chunk, b_chunk)])

    return base_kernel(pid, base_pairs)


def _tc_fuse(rows, a, lora_B_w):
    batch = rows.shape[0]
    block = 2048
    grid = batch // block

    def body(rows_ref, a_ref, bw_ref, o_ref):
        delta = lax.dot_general(
            a_ref[...], bw_ref[...],
            dimension_numbers=(((1,), (1,)), ((), ())),
            preferred_element_type=jnp.float32)
        o_ref[...] = rows_ref[...] + delta

    return pl.pallas_call(
        body,
        grid=(grid,),
        in_specs=[
            pl.BlockSpec((block, DIM), lambda i: (i, 0)),
            pl.BlockSpec((block, RANK), lambda i: (i, 0)),
            pl.BlockSpec((DIM, RANK), lambda i: (0, 0)),
        ],
        out_specs=pl.BlockSpec((block, DIM), lambda i: (i, 0)),
        out_shape=jax.ShapeDtypeStruct((batch, DIM), jnp.float32),
    )(rows, a, lora_B_w)


def kernel(pid, base, lora_A, lora_B_w):
    info = plsc.get_sparse_core_info()
    n_workers = info.num_cores * info.num_subcores
    pid = pid.astype(jnp.int32)
    batch = pid.shape[0]
    a_flat = _sc_gather_lora(pid, lora_A.T, n_workers, info.num_cores)
    base_rm = _tc_transpose(base.T)
    rows = _sc_gather_base(pid, base_rm, n_workers, info.num_cores)
    a = a_flat.reshape(batch, RANK)
    return _tc_fuse(rows, a, lora_B_w)
